# baseline (speedup 1.0000x reference)
import functools
import math

import jax
import jax.numpy as jnp
from jax import lax
from jax.experimental import pallas as pl
from jax.experimental.pallas import tpu as pltpu
from jax.experimental.pallas import tpu_sc as plsc

S = 2048
HID = 1024
NH, NKV, HD = 16, 4, 64
E, TOPK, FFN = 8, 2, 512
SI = 1024
EPS = 1e-6
A = S * TOPK

BT1 = 256
BQ = 512
BK = 512
BT3 = 256
BTF = 128
NB = 40
NPAD = NB * BTF
BTS = 256

NEG = -1e30


def _rope_pair(x, cos, sin, nheads):
    outs = []
    for h in range(nheads):
        xh = x[:, h * HD:(h + 1) * HD]
        rot = jnp.concatenate([-xh[:, HD // 2:], xh[:, :HD // 2]], axis=1)
        outs.append(xh * cos + rot * sin)
    return outs


def _qkv_body(h_ref, ln_ref, wq_ref, wk_ref, wv_ref, cos_ref, sin_ref,
              q_out, k_out, v_out):
    x = h_ref[...]
    var = jnp.mean(x * x, axis=-1, keepdims=True)
    xn = x * lax.rsqrt(var + EPS) * ln_ref[...]
    q = jnp.dot(xn, wq_ref[...], preferred_element_type=jnp.float32)
    k = jnp.dot(xn, wk_ref[...], preferred_element_type=jnp.float32)
    v = jnp.dot(xn, wv_ref[...], preferred_element_type=jnp.float32)
    cos = cos_ref[...]
    sin = sin_ref[...]
    for h, qh in enumerate(_rope_pair(q, cos, sin, NH)):
        q_out[h] = qh
    for h, kh in enumerate(_rope_pair(k, cos, sin, NKV)):
        k_out[h] = kh
    for h in range(NKV):
        v_out[h] = v[:, h * HD:(h + 1) * HD]


def _attn_body(q_ref, k_ref, v_ref, o_ref):
    qb = pl.program_id(1)
    q = q_ref[0]
    s = lax.dot_general(q, k_ref[0], (((1,), (1,)), ((), ())),
                        preferred_element_type=jnp.float32)
    s = s * (1.0 / math.sqrt(HD))
    qpos = lax.broadcasted_iota(jnp.int32, (BQ, S), 0) + qb * BQ
    col = lax.broadcasted_iota(jnp.int32, (BQ, S), 1)
    s = jnp.where(qpos >= col, s, jnp.finfo(jnp.float32).min)
    m = jnp.max(s, axis=1, keepdims=True)
    ex = jnp.exp(s - m)
    p = ex / jnp.sum(ex, axis=1, keepdims=True)
    o_ref[0] = jnp.dot(p, v_ref[0], preferred_element_type=jnp.float32)


def _post_body(a_ref, res_ref, wo_ref, lnp_ref, gw_ref, gb_ref,
               res2_out, flat_out, i1_out, i2_out, w1_out, w2_out):
    a = jnp.concatenate([a_ref[h] for h in range(NH)], axis=1)
    o = jnp.dot(a, wo_ref[...], preferred_element_type=jnp.float32)
    r2 = o + res_ref[...]
    res2_out[...] = r2
    var = jnp.mean(r2 * r2, axis=-1, keepdims=True)
    xn = r2 * lax.rsqrt(var + EPS) * lnp_ref[...]
    flat_out[...] = xn
    logits = jnp.dot(xn, gw_ref[...], preferred_element_type=jnp.float32)
    mx = jnp.max(logits, axis=1, keepdims=True)
    ex = jnp.exp(logits - mx)
    rs = ex / jnp.sum(ex, axis=1, keepdims=True)
    choice = rs + gb_ref[...]
    iot = lax.broadcasted_iota(jnp.int32, (BT3, E), 1)
    m1 = jnp.max(choice, axis=1, keepdims=True)
    i1 = jnp.min(jnp.where(choice == m1, iot, E), axis=1, keepdims=True)
    w1 = jnp.sum(jnp.where(iot == i1, rs, 0.0), axis=1, keepdims=True)
    ch2 = jnp.where(iot == i1, NEG, choice)
    m2 = jnp.max(ch2, axis=1, keepdims=True)
    i2 = jnp.min(jnp.where(ch2 == m2, iot, E), axis=1, keepdims=True)
    w2 = jnp.sum(jnp.where(iot == i2, rs, 0.0), axis=1, keepdims=True)
    den = w1 + w2 + 1e-20
    i1_out[...] = i1
    i2_out[...] = i2
    w1_out[...] = w1 / den
    w2_out[...] = w2 / den


def _ffn_body(be_ref, xs_ref, wg_ref, wd_ref, ys_ref):
    x = xs_ref[...]
    gu = jnp.dot(x, wg_ref[0], preferred_element_type=jnp.float32)
    g = gu[:, :FFN]
    u = gu[:, FFN:]
    act = g * jax.nn.sigmoid(g) * u
    ys_ref[...] = jnp.dot(act, wd_ref[0], preferred_element_type=jnp.float32)


def _shared_body(x_ref, wsgu_ref, wsd_ref, out_ref):
    x = x_ref[...]
    sgu = jnp.dot(x, wsgu_ref[...], preferred_element_type=jnp.float32)
    sg = sgu[:, :SI]
    su = sgu[:, SI:]
    act = sg * jax.nn.sigmoid(sg) * su
    out_ref[...] = jnp.dot(act, wsd_ref[...],
                           preferred_element_type=jnp.float32)


def _add3_body(a_ref, b_ref, c_ref, w1_ref, w2_ref, out_ref):
    out_ref[...] = (a_ref[...] + w1_ref[...] * b_ref[0]
                    + w2_ref[...] * c_ref[0])


def _sc_gather_rows(table, idx, n_rows, chunk):
    nw = 32
    b_per_w = n_rows // nw
    nch = b_per_w // chunk
    mesh = plsc.VectorSubcoreMesh(core_axis_name="c", subcore_axis_name="s")

    @functools.partial(
        pl.kernel, mesh=mesh,
        out_type=jax.ShapeDtypeStruct((n_rows, HID), jnp.float32),
        scratch_types=[
            pltpu.VMEM((b_per_w,), jnp.int32),
            pltpu.VMEM((chunk, HID), jnp.float32),
            pltpu.VMEM((chunk, HID), jnp.float32),
            pltpu.SemaphoreType.DMA,
            pltpu.SemaphoreType.DMA,
        ],
    )
    def gk(table_hbm, idx_hbm, out_hbm, idx_v, buf0, buf1, sem0, sem1):
        wid = lax.axis_index("s") * 2 + lax.axis_index("c")
        base = wid * b_per_w
        pltpu.sync_copy(idx_hbm.at[pl.ds(base, b_per_w)], idx_v)
        bufs = (buf0, buf1)
        sems = (sem0, sem1)
        dmas = [None, None]
        dmas[0] = pltpu.async_copy(
            table_hbm.at[idx_v.at[pl.ds(0, chunk)]], bufs[0], sems[0])
        for c in range(nch):
            if c + 1 < nch:
                dmas[(c + 1) % 2] = pltpu.async_copy(
                    table_hbm.at[idx_v.at[pl.ds((c + 1) * chunk, chunk)]],
                    bufs[(c + 1) % 2], sems[(c + 1) % 2])
            dmas[c % 2].wait()
            pltpu.sync_copy(bufs[c % 2],
                            out_hbm.at[pl.ds(base + c * chunk, chunk)])

    return gk(table, idx)


def _routing_metadata(i1, i2):
    ids = jnp.concatenate([i1, i2], axis=1).reshape(-1)
    onehot = ids[:, None] == jnp.arange(E, dtype=jnp.int32)[None, :]
    csum = jnp.cumsum(onehot.astype(jnp.int32), axis=0)
    rank = jnp.sum(jnp.where(onehot, csum - 1, 0), axis=1)
    sizes = csum[-1]
    nblk = (sizes + BTF - 1) // BTF
    bcum = jnp.cumsum(nblk)
    pad_start = (bcum - nblk) * BTF
    inv = jnp.sum(jnp.where(onehot, pad_start[None, :], 0), axis=1) + rank
    bidx = jnp.arange(NB, dtype=jnp.int32)
    bexp = jnp.sum((bidx[:, None] >= bcum[None, :]).astype(jnp.int32), axis=1)
    bexp = jnp.where(bexp < E, bexp, 0)
    toks = jnp.arange(A, dtype=jnp.int32) // TOPK
    tok_idx = (jnp.arange(NPAD, dtype=jnp.int32) % S).at[inv].set(toks)
    pos01 = inv.reshape(S, TOPK)
    gidx2 = jnp.concatenate([pos01[:, 0], pos01[:, 1]]).astype(jnp.int32)
    return bexp, tok_idx, gidx2


def kernel(hidden_states, cos, sin, Wq, Wk, Wv, Wo, gate_W, gate_bias,
           W_gate_up, W_down, Ws_gate_up, Ws_down, ln_in_w, ln_post_w):
    x2d = hidden_states.reshape(S, HID)
    ln_in = ln_in_w.reshape(1, HID)
    ln_post = ln_post_w.reshape(1, HID)
    gwT = gate_W.T
    gb = gate_bias.reshape(1, E)

    q, k, v = pl.pallas_call(
        _qkv_body,
        grid=(S // BT1,),
        in_specs=[
            pl.BlockSpec((BT1, HID), lambda i: (i, 0)),
            pl.BlockSpec((1, HID), lambda i: (0, 0)),
            pl.BlockSpec((HID, NH * HD), lambda i: (0, 0)),
            pl.BlockSpec((HID, NKV * HD), lambda i: (0, 0)),
            pl.BlockSpec((HID, NKV * HD), lambda i: (0, 0)),
            pl.BlockSpec((BT1, HD), lambda i: (i, 0)),
            pl.BlockSpec((BT1, HD), lambda i: (i, 0)),
        ],
        out_specs=[
            pl.BlockSpec((NH, BT1, HD), lambda i: (0, i, 0)),
            pl.BlockSpec((NKV, BT1, HD), lambda i: (0, i, 0)),
            pl.BlockSpec((NKV, BT1, HD), lambda i: (0, i, 0)),
        ],
        out_shape=[
            jax.ShapeDtypeStruct((NH, S, HD), jnp.float32),
            jax.ShapeDtypeStruct((NKV, S, HD), jnp.float32),
            jax.ShapeDtypeStruct((NKV, S, HD), jnp.float32),
        ],
    )(x2d, ln_in, Wq, Wk, Wv, cos, sin)

    rep = NH // NKV
    attn = pl.pallas_call(
        _attn_body,
        grid=(NH, S // BQ),
        in_specs=[
            pl.BlockSpec((1, BQ, HD), lambda h, i: (h, i, 0)),
            pl.BlockSpec((1, S, HD), lambda h, i: (h // rep, 0, 0)),
            pl.BlockSpec((1, S, HD), lambda h, i: (h // rep, 0, 0)),
        ],
        out_specs=pl.BlockSpec((1, BQ, HD), lambda h, i: (h, i, 0)),
        out_shape=jax.ShapeDtypeStruct((NH, S, HD), jnp.float32),
        compiler_params=pltpu.CompilerParams(
            dimension_semantics=("arbitrary", "arbitrary")),
    )(q, k, v)

    res2, flat, i1, i2, w1, w2 = pl.pallas_call(
        _post_body,
        grid=(S // BT3,),
        in_specs=[
            pl.BlockSpec((NH, BT3, HD), lambda i: (0, i, 0)),
            pl.BlockSpec((BT3, HID), lambda i: (i, 0)),
            pl.BlockSpec((NH * HD, HID), lambda i: (0, 0)),
            pl.BlockSpec((1, HID), lambda i: (0, 0)),
            pl.BlockSpec((HID, E), lambda i: (0, 0)),
            pl.BlockSpec((1, E), lambda i: (0, 0)),
        ],
        out_specs=[
            pl.BlockSpec((BT3, HID), lambda i: (i, 0)),
            pl.BlockSpec((BT3, HID), lambda i: (i, 0)),
            pl.BlockSpec((BT3, 1), lambda i: (i, 0)),
            pl.BlockSpec((BT3, 1), lambda i: (i, 0)),
            pl.BlockSpec((BT3, 1), lambda i: (i, 0)),
            pl.BlockSpec((BT3, 1), lambda i: (i, 0)),
        ],
        out_shape=[
            jax.ShapeDtypeStruct((S, HID), jnp.float32),
            jax.ShapeDtypeStruct((S, HID), jnp.float32),
            jax.ShapeDtypeStruct((S, 1), jnp.int32),
            jax.ShapeDtypeStruct((S, 1), jnp.int32),
            jax.ShapeDtypeStruct((S, 1), jnp.float32),
            jax.ShapeDtypeStruct((S, 1), jnp.float32),
        ],
    )(attn, x2d, Wo, ln_post, gwT, gb)

    bexp, tok_idx, gidx2 = _routing_metadata(i1, i2)

    xs = _sc_gather_rows(flat, tok_idx, NPAD, 32)
    ys = pl.pallas_call(
        _ffn_body,
        grid_spec=pltpu.PrefetchScalarGridSpec(
            num_scalar_prefetch=1,
            grid=(NB,),
            in_specs=[
                pl.BlockSpec((BTF, HID), lambda b, be: (b, 0)),
                pl.BlockSpec((1, HID, 2 * FFN), lambda b, be: (be[b], 0, 0)),
                pl.BlockSpec((1, FFN, HID), lambda b, be: (be[b], 0, 0)),
            ],
            out_specs=pl.BlockSpec((BTF, HID), lambda b, be: (b, 0)),
        ),
        out_shape=jax.ShapeDtypeStruct((NPAD, HID), jnp.float32),
    )(bexp, xs, W_gate_up, W_down)

    shared = pl.pallas_call(
        _shared_body,
        grid=(S // BTS,),
        in_specs=[
            pl.BlockSpec((BTS, HID), lambda i: (i, 0)),
            pl.BlockSpec((HID, 2 * SI), lambda i: (0, 0)),
            pl.BlockSpec((SI, HID), lambda i: (0, 0)),
        ],
        out_specs=pl.BlockSpec((BTS, HID), lambda i: (i, 0)),
        out_shape=jax.ShapeDtypeStruct((S, HID), jnp.float32),
    )(flat, Ws_gate_up, Ws_down)

    yg = _sc_gather_rows(ys, gidx2, A, 32).reshape(TOPK, S, HID)
    out = pl.pallas_call(
        _add3_body,
        grid=(S // 512,),
        in_specs=[
            pl.BlockSpec((512, HID), lambda i: (i, 0)),
            pl.BlockSpec((1, 512, HID), lambda i: (0, i, 0)),
            pl.BlockSpec((1, 512, HID), lambda i: (1, i, 0)),
            pl.BlockSpec((512, 1), lambda i: (i, 0)),
            pl.BlockSpec((512, 1), lambda i: (i, 0)),
        ],
        out_specs=pl.BlockSpec((512, HID), lambda i: (i, 0)),
        out_shape=jax.ShapeDtypeStruct((S, HID), jnp.float32),
    )(shared, yg, yg, w1, w2)

    return out.reshape(1, S, HID), res2.reshape(1, S, HID)


# speedup vs baseline: 1.1781x; 1.1781x over previous
import functools
import math

import jax
import jax.numpy as jnp
from jax import lax
from jax.experimental import pallas as pl
from jax.experimental.pallas import tpu as pltpu
from jax.experimental.pallas import tpu_sc as plsc

S = 2048
HID = 1024
NH, NKV, HD = 16, 4, 64
E, TOPK, FFN = 8, 2, 512
SI = 1024
EPS = 1e-6
A = S * TOPK

BT1 = 256
BQ = 512
BK = 512
BT3 = 256
BTF = 128
NB = 40
NPAD = NB * BTF
BTS = 256

NEG = -1e30


def _rope_pair(x, cos, sin, nheads):
    outs = []
    for h in range(nheads):
        xh = x[:, h * HD:(h + 1) * HD]
        rot = jnp.concatenate([-xh[:, HD // 2:], xh[:, :HD // 2]], axis=1)
        outs.append(xh * cos + rot * sin)
    return outs


def _qkv_body(h_ref, ln_ref, wq_ref, wk_ref, wv_ref, cos_ref, sin_ref,
              q_out, k_out, v_out):
    x = h_ref[...]
    var = jnp.mean(x * x, axis=-1, keepdims=True)
    xn = x * (1.0 / jnp.sqrt(var + EPS)) * ln_ref[...]
    q = jnp.dot(xn, wq_ref[...], preferred_element_type=jnp.float32)
    k = jnp.dot(xn, wk_ref[...], preferred_element_type=jnp.float32)
    v = jnp.dot(xn, wv_ref[...], preferred_element_type=jnp.float32)
    cos = cos_ref[...]
    sin = sin_ref[...]
    for h, qh in enumerate(_rope_pair(q, cos, sin, NH)):
        q_out[h] = qh
    for h, kh in enumerate(_rope_pair(k, cos, sin, NKV)):
        k_out[h] = kh
    for h in range(NKV):
        v_out[h] = v[:, h * HD:(h + 1) * HD]


def _attn_block_body(qb, L):
    def body(q_ref, k_ref, v_ref, o_ref):
        q = q_ref[0]
        s = lax.dot_general(q, k_ref[0], (((1,), (1,)), ((), ())),
                            preferred_element_type=jnp.float32)
        s = s * (1.0 / math.sqrt(HD))
        qpos = lax.broadcasted_iota(jnp.int32, (BQ, L), 0) + qb * BQ
        col = lax.broadcasted_iota(jnp.int32, (BQ, L), 1)
        s = jnp.where(qpos >= col, s, jnp.finfo(jnp.float32).min)
        m = jnp.max(s, axis=1, keepdims=True)
        ex = jnp.exp(s - m)
        p = ex / jnp.sum(ex, axis=1, keepdims=True)
        o_ref[0] = jnp.dot(p, v_ref[0], preferred_element_type=jnp.float32)
    return body


def _post_body(a_ref, res_ref, wo_ref, lnp_ref, gw_ref, gb_ref,
               res2_out, flat_out, i1_out, i2_out, w1_out, w2_out):
    a = jnp.concatenate([a_ref[h] for h in range(NH)], axis=1)
    o = jnp.dot(a, wo_ref[...], preferred_element_type=jnp.float32)
    r2 = o + res_ref[...]
    res2_out[...] = r2
    var = jnp.mean(r2 * r2, axis=-1, keepdims=True)
    xn = r2 * (1.0 / jnp.sqrt(var + EPS)) * lnp_ref[...]
    flat_out[...] = xn
    logits = jnp.dot(xn, gw_ref[...], preferred_element_type=jnp.float32)
    mx = jnp.max(logits, axis=1, keepdims=True)
    ex = jnp.exp(logits - mx)
    rs = ex / jnp.sum(ex, axis=1, keepdims=True)
    choice = rs + gb_ref[...]
    iot = lax.broadcasted_iota(jnp.int32, (BT3, E), 1)
    m1 = jnp.max(choice, axis=1, keepdims=True)
    i1 = jnp.min(jnp.where(choice == m1, iot, E), axis=1, keepdims=True)
    w1 = jnp.sum(jnp.where(iot == i1, rs, 0.0), axis=1, keepdims=True)
    ch2 = jnp.where(iot == i1, NEG, choice)
    m2 = jnp.max(ch2, axis=1, keepdims=True)
    i2 = jnp.min(jnp.where(ch2 == m2, iot, E), axis=1, keepdims=True)
    w2 = jnp.sum(jnp.where(iot == i2, rs, 0.0), axis=1, keepdims=True)
    den = w1 + w2 + 1e-20
    i1_out[...] = i1
    i2_out[...] = i2
    w1_out[...] = w1 / den
    w2_out[...] = w2 / den


def _ffn_body(be_ref, xs_ref, wg_ref, wd_ref, ys_ref):
    x = xs_ref[...]
    gu = jnp.dot(x, wg_ref[0], preferred_element_type=jnp.float32)
    g = gu[:, :FFN]
    u = gu[:, FFN:]
    act = g * jax.nn.sigmoid(g) * u
    ys_ref[...] = jnp.dot(act, wd_ref[0], preferred_element_type=jnp.float32)


def _shared_body(x_ref, wsgu_ref, wsd_ref, out_ref):
    x = x_ref[...]
    sgu = jnp.dot(x, wsgu_ref[...], preferred_element_type=jnp.float32)
    sg = sgu[:, :SI]
    su = sgu[:, SI:]
    act = sg * jax.nn.sigmoid(sg) * su
    out_ref[...] = jnp.dot(act, wsd_ref[...],
                           preferred_element_type=jnp.float32)


def _add3_body(a_ref, b_ref, c_ref, w1_ref, w2_ref, out_ref):
    out_ref[...] = (a_ref[...] + w1_ref[...] * b_ref[0]
                    + w2_ref[...] * c_ref[0])


def _sc_gather_rows(table, idx, n_rows, chunk):
    nw = 32
    b_per_w = n_rows // nw
    nch = b_per_w // chunk
    mesh = plsc.VectorSubcoreMesh(core_axis_name="c", subcore_axis_name="s")

    @functools.partial(
        pl.kernel, mesh=mesh,
        out_type=jax.ShapeDtypeStruct((n_rows, HID), jnp.float32),
        scratch_types=[
            pltpu.VMEM((b_per_w,), jnp.int32),
            pltpu.VMEM((chunk, HID), jnp.float32),
            pltpu.VMEM((chunk, HID), jnp.float32),
            pltpu.SemaphoreType.DMA,
            pltpu.SemaphoreType.DMA,
        ],
    )
    def gk(table_hbm, idx_hbm, out_hbm, idx_v, buf0, buf1, sem0, sem1):
        wid = lax.axis_index("s") * 2 + lax.axis_index("c")
        base = wid * b_per_w
        pltpu.sync_copy(idx_hbm.at[pl.ds(base, b_per_w)], idx_v)
        bufs = (buf0, buf1)
        sems = (sem0, sem1)
        dmas = [None, None]
        dmas[0] = pltpu.async_copy(
            table_hbm.at[idx_v.at[pl.ds(0, chunk)]], bufs[0], sems[0])
        for c in range(nch):
            if c + 1 < nch:
                dmas[(c + 1) % 2] = pltpu.async_copy(
                    table_hbm.at[idx_v.at[pl.ds((c + 1) * chunk, chunk)]],
                    bufs[(c + 1) % 2], sems[(c + 1) % 2])
            dmas[c % 2].wait()
            pltpu.sync_copy(bufs[c % 2],
                            out_hbm.at[pl.ds(base + c * chunk, chunk)])

    return gk(table, idx)


def _routing_metadata(i1, i2):
    ids = jnp.concatenate([i1, i2], axis=1).reshape(-1)
    onehot = ids[:, None] == jnp.arange(E, dtype=jnp.int32)[None, :]
    csum = jnp.cumsum(onehot.astype(jnp.int32), axis=0)
    rank = jnp.sum(jnp.where(onehot, csum - 1, 0), axis=1)
    sizes = csum[-1]
    nblk = (sizes + BTF - 1) // BTF
    bcum = jnp.cumsum(nblk)
    pad_start = (bcum - nblk) * BTF
    inv = jnp.sum(jnp.where(onehot, pad_start[None, :], 0), axis=1) + rank
    bidx = jnp.arange(NB, dtype=jnp.int32)
    bexp = jnp.sum((bidx[:, None] >= bcum[None, :]).astype(jnp.int32), axis=1)
    bexp = jnp.where(bexp < E, bexp, 0)
    toks = jnp.arange(A, dtype=jnp.int32) // TOPK
    tok_idx = (jnp.arange(NPAD, dtype=jnp.int32) % S).at[inv].set(toks)
    pos01 = inv.reshape(S, TOPK)
    gidx2 = jnp.concatenate([pos01[:, 0], pos01[:, 1]]).astype(jnp.int32)
    return bexp, tok_idx, gidx2


def kernel(hidden_states, cos, sin, Wq, Wk, Wv, Wo, gate_W, gate_bias,
           W_gate_up, W_down, Ws_gate_up, Ws_down, ln_in_w, ln_post_w):
    x2d = hidden_states.reshape(S, HID)
    ln_in = ln_in_w.reshape(1, HID)
    ln_post = ln_post_w.reshape(1, HID)
    gwT = gate_W.T
    gb = gate_bias.reshape(1, E)

    q, k, v = pl.pallas_call(
        _qkv_body,
        grid=(S // BT1,),
        in_specs=[
            pl.BlockSpec((BT1, HID), lambda i: (i, 0)),
            pl.BlockSpec((1, HID), lambda i: (0, 0)),
            pl.BlockSpec((HID, NH * HD), lambda i: (0, 0)),
            pl.BlockSpec((HID, NKV * HD), lambda i: (0, 0)),
            pl.BlockSpec((HID, NKV * HD), lambda i: (0, 0)),
            pl.BlockSpec((BT1, HD), lambda i: (i, 0)),
            pl.BlockSpec((BT1, HD), lambda i: (i, 0)),
        ],
        out_specs=[
            pl.BlockSpec((NH, BT1, HD), lambda i: (0, i, 0)),
            pl.BlockSpec((NKV, BT1, HD), lambda i: (0, i, 0)),
            pl.BlockSpec((NKV, BT1, HD), lambda i: (0, i, 0)),
        ],
        out_shape=[
            jax.ShapeDtypeStruct((NH, S, HD), jnp.float32),
            jax.ShapeDtypeStruct((NKV, S, HD), jnp.float32),
            jax.ShapeDtypeStruct((NKV, S, HD), jnp.float32),
        ],
    )(x2d, ln_in, Wq, Wk, Wv, cos, sin)

    rep = NH // NKV
    attn_blocks = []
    for qb in range(S // BQ):
        L = (qb + 1) * BQ
        attn_blocks.append(pl.pallas_call(
            _attn_block_body(qb, L),
            grid=(NH,),
            in_specs=[
                pl.BlockSpec((1, BQ, HD), lambda h, _qb=qb: (h, _qb, 0)),
                pl.BlockSpec((1, L, HD), lambda h: (h // rep, 0, 0)),
                pl.BlockSpec((1, L, HD), lambda h: (h // rep, 0, 0)),
            ],
            out_specs=pl.BlockSpec((1, BQ, HD), lambda h: (h, 0, 0)),
            out_shape=jax.ShapeDtypeStruct((NH, BQ, HD), jnp.float32),
        )(q, k, v))
    attn = jnp.concatenate(attn_blocks, axis=1)

    res2, flat, i1, i2, w1, w2 = pl.pallas_call(
        _post_body,
        grid=(S // BT3,),
        in_specs=[
            pl.BlockSpec((NH, BT3, HD), lambda i: (0, i, 0)),
            pl.BlockSpec((BT3, HID), lambda i: (i, 0)),
            pl.BlockSpec((NH * HD, HID), lambda i: (0, 0)),
            pl.BlockSpec((1, HID), lambda i: (0, 0)),
            pl.BlockSpec((HID, E), lambda i: (0, 0)),
            pl.BlockSpec((1, E), lambda i: (0, 0)),
        ],
        out_specs=[
            pl.BlockSpec((BT3, HID), lambda i: (i, 0)),
            pl.BlockSpec((BT3, HID), lambda i: (i, 0)),
            pl.BlockSpec((BT3, 1), lambda i: (i, 0)),
            pl.BlockSpec((BT3, 1), lambda i: (i, 0)),
            pl.BlockSpec((BT3, 1), lambda i: (i, 0)),
            pl.BlockSpec((BT3, 1), lambda i: (i, 0)),
        ],
        out_shape=[
            jax.ShapeDtypeStruct((S, HID), jnp.float32),
            jax.ShapeDtypeStruct((S, HID), jnp.float32),
            jax.ShapeDtypeStruct((S, 1), jnp.int32),
            jax.ShapeDtypeStruct((S, 1), jnp.int32),
            jax.ShapeDtypeStruct((S, 1), jnp.float32),
            jax.ShapeDtypeStruct((S, 1), jnp.float32),
        ],
    )(attn, x2d, Wo, ln_post, gwT, gb)

    bexp, tok_idx, gidx2 = _routing_metadata(i1, i2)

    xs = _sc_gather_rows(flat, tok_idx, NPAD, 32)
    ys = pl.pallas_call(
        _ffn_body,
        grid_spec=pltpu.PrefetchScalarGridSpec(
            num_scalar_prefetch=1,
            grid=(NB,),
            in_specs=[
                pl.BlockSpec((BTF, HID), lambda b, be: (b, 0)),
                pl.BlockSpec((1, HID, 2 * FFN), lambda b, be: (be[b], 0, 0)),
                pl.BlockSpec((1, FFN, HID), lambda b, be: (be[b], 0, 0)),
            ],
            out_specs=pl.BlockSpec((BTF, HID), lambda b, be: (b, 0)),
        ),
        out_shape=jax.ShapeDtypeStruct((NPAD, HID), jnp.float32),
    )(bexp, xs, W_gate_up, W_down)

    shared = pl.pallas_call(
        _shared_body,
        grid=(S // BTS,),
        in_specs=[
            pl.BlockSpec((BTS, HID), lambda i: (i, 0)),
            pl.BlockSpec((HID, 2 * SI), lambda i: (0, 0)),
            pl.BlockSpec((SI, HID), lambda i: (0, 0)),
        ],
        out_specs=pl.BlockSpec((BTS, HID), lambda i: (i, 0)),
        out_shape=jax.ShapeDtypeStruct((S, HID), jnp.float32),
    )(flat, Ws_gate_up, Ws_down)

    yg = _sc_gather_rows(ys, gidx2, A, 32).reshape(TOPK, S, HID)
    out = pl.pallas_call(
        _add3_body,
        grid=(S // 512,),
        in_specs=[
            pl.BlockSpec((512, HID), lambda i: (i, 0)),
            pl.BlockSpec((1, 512, HID), lambda i: (0, i, 0)),
            pl.BlockSpec((1, 512, HID), lambda i: (1, i, 0)),
            pl.BlockSpec((512, 1), lambda i: (i, 0)),
            pl.BlockSpec((512, 1), lambda i: (i, 0)),
        ],
        out_specs=pl.BlockSpec((512, HID), lambda i: (i, 0)),
        out_shape=jax.ShapeDtypeStruct((S, HID), jnp.float32),
    )(shared, yg, yg, w1, w2)

    return out.reshape(1, S, HID), res2.reshape(1, S, HID)
